# baseline (speedup 1.0000x reference)
import jax
import jax.numpy as jnp
from jax.experimental import pallas as pl
from jax.experimental.pallas import tpu as pltpu

S, P, H, EMB = 128, 16, 64, 64
D1, D2 = 512, 1024
NS = 128


def _pool_kernel(pos_ref, h_ref, Wsp_ref, bsp_ref,
                 Wp1_ref, bp1_ref, gp1_ref, btp1_ref,
                 Wp2_ref, bp2_ref, gp2_ref, btp2_ref,
                 out_ref, A_s, Df_s, c_s, W2_s, b2_s):
    @pl.when(pl.program_id(0) == 0)
    def _prep():
        inv = 1.0 / jnp.sqrt(1.0 + 1e-5)
        s1 = gp1_ref[...] * inv
        W1t = Wp1_ref[:EMB] * s1
        A_s[...] = 0.05 * jnp.dot(Wsp_ref[...], W1t,
                                  preferred_element_type=jnp.float32)
        Df_s[...] = (0.05 * (Wp1_ref[EMB:] * s1)).astype(jnp.bfloat16)
        c_s[...] = (0.05 * jnp.dot(bsp_ref[...], W1t,
                                   preferred_element_type=jnp.float32)
                    + bp1_ref[...] * s1 + btp1_ref[...])
        s2 = gp2_ref[...] * inv
        W2_s[...] = (Wp2_ref[...] * s2).astype(jnp.bfloat16)
        b2_s[...] = bp2_ref[...] * s2 + btp2_ref[...]

    pos = pos_ref[...]
    h = h_ref[...]
    A = A_s[...]
    uv = pos[:, 0:1] * A[0:1, :] + pos[:, 1:2] * A[1:2, :]
    u = uv + jnp.dot(h.astype(jnp.bfloat16), Df_s[...],
                     preferred_element_type=jnp.float32)
    u = (u + c_s[...]).astype(jnp.bfloat16)
    v = uv.astype(jnp.bfloat16)
    u3 = u.reshape(NS, P, D1)
    v3 = v.reshape(NS, P, D1)
    W2 = W2_s[...]
    acc = None
    for j in range(P):
        x1 = jnp.maximum(u3[:, j:j + 1, :] - v3, 0).reshape(NS * P, D1)
        zj = jnp.dot(x1, W2, preferred_element_type=jnp.float32)
        acc = zj if acc is None else jnp.maximum(acc, zj)
    out_ref[...] = jnp.maximum(acc + b2_s[...], 0.0)


@jax.jit
def kernel(h_states, seq_start_end, end_pos, vx, vy,
           W_sp, b_sp, W_vel, b_vel,
           Wa1, ba1, ga1, bta1, Wa2, ba2, ga2, bta2,
           Wp1, bp1, gp1, btp1, Wp2, bp2, gp2, btp2):
    B = end_pos.shape[0]
    h = h_states.reshape(B, H)
    blk = NS * P
    row = lambda a: a.reshape(1, -1)
    whole = lambda shp: pl.BlockSpec(shp, lambda i: (0, 0))
    out = pl.pallas_call(
        _pool_kernel,
        grid=(S // NS,),
        in_specs=[
            pl.BlockSpec((blk, 2), lambda i: (i, 0)),
            pl.BlockSpec((blk, H), lambda i: (i, 0)),
            whole((2, EMB)), whole((1, EMB)),
            whole((2 * EMB, D1)), whole((1, D1)), whole((1, D1)), whole((1, D1)),
            whole((D1, D2)), whole((1, D2)), whole((1, D2)), whole((1, D2)),
        ],
        out_specs=pl.BlockSpec((blk, D2), lambda i: (i, 0)),
        out_shape=jax.ShapeDtypeStruct((B, D2), jnp.float32),
        scratch_shapes=[
            pltpu.VMEM((2, D1), jnp.float32),
            pltpu.VMEM((EMB, D1), jnp.bfloat16),
            pltpu.VMEM((1, D1), jnp.float32),
            pltpu.VMEM((D1, D2), jnp.bfloat16),
            pltpu.VMEM((1, D2), jnp.float32),
        ],
    )(end_pos, h, W_sp, row(b_sp), Wp1, row(bp1), row(gp1), row(btp1),
      Wp2, row(bp2), row(gp2), row(btp2))
    return out


# speedup vs baseline: 1.0508x; 1.0508x over previous
import jax
import jax.numpy as jnp
from jax.experimental import pallas as pl
from jax.experimental.pallas import tpu as pltpu

S, P, H, EMB = 128, 16, 64, 64
D1, D2 = 512, 1024
NS = 64


def _pool_kernel(pos_ref, h_ref, Wsp_ref, bsp_ref,
                 Wp1_ref, bp1_ref, gp1_ref, btp1_ref,
                 Wp2_ref, bp2_ref, gp2_ref, btp2_ref,
                 out_ref, A_s, Df_s, c_s, W2_s, b2_s):
    @pl.when(pl.program_id(0) == 0)
    def _prep():
        inv = 1.0 / jnp.sqrt(1.0 + 1e-5)
        s1 = gp1_ref[...] * inv
        W1t = Wp1_ref[:EMB] * s1
        A_s[...] = 0.05 * jnp.dot(Wsp_ref[...], W1t,
                                  preferred_element_type=jnp.float32)
        Df_s[...] = (0.05 * (Wp1_ref[EMB:] * s1)).astype(jnp.bfloat16)
        c_s[...] = (0.05 * jnp.dot(bsp_ref[...], W1t,
                                   preferred_element_type=jnp.float32)
                    + bp1_ref[...] * s1 + btp1_ref[...])
        s2 = gp2_ref[...] * inv
        W2_s[...] = (Wp2_ref[...] * s2).astype(jnp.bfloat16)
        b2_s[...] = bp2_ref[...] * s2 + btp2_ref[...]

    pos = pos_ref[...]
    h = h_ref[...]
    A = A_s[...]
    uv = pos[:, 0:1] * A[0:1, :] + pos[:, 1:2] * A[1:2, :]
    u = uv + jnp.dot(h.astype(jnp.bfloat16), Df_s[...],
                     preferred_element_type=jnp.float32)
    u = (u + c_s[...]).astype(jnp.bfloat16)
    v = uv.astype(jnp.bfloat16)
    u3 = u.reshape(NS, P, D1)
    v3 = v.reshape(NS, P, D1)
    W2 = W2_s[...]
    acc = None
    for j in range(P):
        x1 = jnp.maximum(u3[:, j:j + 1, :] - v3, 0).reshape(NS * P, D1)
        zj = jnp.dot(x1, W2, preferred_element_type=jnp.float32)
        acc = zj if acc is None else jnp.maximum(acc, zj)
    out_ref[...] = jnp.maximum(acc + b2_s[...], 0.0)


@jax.jit
def kernel(h_states, seq_start_end, end_pos, vx, vy,
           W_sp, b_sp, W_vel, b_vel,
           Wa1, ba1, ga1, bta1, Wa2, ba2, ga2, bta2,
           Wp1, bp1, gp1, btp1, Wp2, bp2, gp2, btp2):
    B = end_pos.shape[0]
    h = h_states.reshape(B, H)
    blk = NS * P
    row = lambda a: a.reshape(1, -1)
    whole = lambda shp: pl.BlockSpec(shp, lambda i: (0, 0))
    out = pl.pallas_call(
        _pool_kernel,
        grid=(S // NS,),
        in_specs=[
            pl.BlockSpec((blk, 2), lambda i: (i, 0)),
            pl.BlockSpec((blk, H), lambda i: (i, 0)),
            whole((2, EMB)), whole((1, EMB)),
            whole((2 * EMB, D1)), whole((1, D1)), whole((1, D1)), whole((1, D1)),
            whole((D1, D2)), whole((1, D2)), whole((1, D2)), whole((1, D2)),
        ],
        out_specs=pl.BlockSpec((blk, D2), lambda i: (i, 0)),
        out_shape=jax.ShapeDtypeStruct((B, D2), jnp.float32),
        scratch_shapes=[
            pltpu.VMEM((2, D1), jnp.float32),
            pltpu.VMEM((EMB, D1), jnp.bfloat16),
            pltpu.VMEM((1, D1), jnp.float32),
            pltpu.VMEM((D1, D2), jnp.bfloat16),
            pltpu.VMEM((1, D2), jnp.float32),
        ],
    )(end_pos, h, W_sp, row(b_sp), Wp1, row(bp1), row(gp1), row(btp1),
      Wp2, row(bp2), row(gp2), row(btp2))
    return out
